# Initial kernel scaffold; baseline (speedup 1.0000x reference)
import jax
import jax.numpy as jnp
from jax.experimental import pallas as pl


def kernel(cls_logits, box_preds, intention_logits, anchors, gt_boxes_xywha, gt_intentions):
    raise NotImplementedError("write your pallas kernel here")


# speedup vs baseline: 26.3040x; 26.3040x over previous
import jax
import jax.numpy as jnp
from jax import lax
from jax.experimental import pallas as pl
from jax.experimental.pallas import tpu as pltpu

_IOU_THR = 0.6
_NEG_THR = 0.45
_ALPHA = 0.25
_BETA = 1.0 / 9.0
_CLS_W = 1.0
_BOX_W = 1.0
_INT_W = 0.5

_N = 20000
_NP = 20480
_S, _L = 160, 128
_G = 50
_NI = 8
_B = 4
_EPS = 1e-6


def _loss_kernel(cls_ref, box_ref, il_ref, anc_ref, gt_ref, out_ref,
                 max_ref, wx_ref, wy_ref, ww_ref, wh_ref, wa_ref, wi_ref,
                 forced_ref):
    f32 = jnp.float32

    ax = anc_ref[0][None]
    ay = anc_ref[1][None]
    aw = anc_ref[2][None]
    ah = anc_ref[3][None]
    aa = anc_ref[4][None]
    ax1 = ax - aw * 0.5
    ay1 = ay - ah * 0.5
    ax2 = ax + aw * 0.5
    ay2 = ay + ah * 0.5
    area_a = (ax2 - ax1) * (ay2 - ay1)

    idx_lin = (
        lax.broadcasted_iota(jnp.int32, (1, _S, _L), 1) * _L
        + lax.broadcasted_iota(jnp.int32, (1, _S, _L), 2)
    )

    zero = jnp.zeros((_B, _S, _L), dtype=f32)
    max_ref[...] = jnp.full((_B, _S, _L), -1.0, dtype=f32)
    wx_ref[...] = zero
    wy_ref[...] = zero
    ww_ref[...] = zero
    wh_ref[...] = zero
    wa_ref[...] = zero
    wi_ref[...] = zero
    forced_ref[...] = zero

    def gstep(g, carry):
        gp = gt_ref[g]
        gx = gp[0].reshape(_B, 1, 1)
        gy = gp[1].reshape(_B, 1, 1)
        gw = gp[2].reshape(_B, 1, 1)
        gh = gp[3].reshape(_B, 1, 1)
        ga = gp[4].reshape(_B, 1, 1)
        gi = gp[5].reshape(_B, 1, 1)
        gx1 = gx - gw * 0.5
        gy1 = gy - gh * 0.5
        gx2 = gx + gw * 0.5
        gy2 = gy + gh * 0.5
        area_g = (gx2 - gx1) * (gy2 - gy1)

        ix1 = jnp.maximum(ax1, gx1)
        iy1 = jnp.maximum(ay1, gy1)
        ix2 = jnp.minimum(ax2, gx2)
        iy2 = jnp.minimum(ay2, gy2)
        iw = jnp.maximum(ix2 - ix1, 0.0)
        ih = jnp.maximum(iy2 - iy1, 0.0)
        inter = iw * ih
        iou_g = inter / (area_a + area_g - inter + _EPS)

        better = iou_g > max_ref[...]
        max_ref[...] = jnp.where(better, iou_g, max_ref[...])
        wx_ref[...] = jnp.where(better, jnp.broadcast_to(gx, (_B, _S, _L)), wx_ref[...])
        wy_ref[...] = jnp.where(better, jnp.broadcast_to(gy, (_B, _S, _L)), wy_ref[...])
        ww_ref[...] = jnp.where(better, jnp.broadcast_to(gw, (_B, _S, _L)), ww_ref[...])
        wh_ref[...] = jnp.where(better, jnp.broadcast_to(gh, (_B, _S, _L)), wh_ref[...])
        wa_ref[...] = jnp.where(better, jnp.broadcast_to(ga, (_B, _S, _L)), wa_ref[...])
        wi_ref[...] = jnp.where(better, jnp.broadcast_to(gi, (_B, _S, _L)), wi_ref[...])

        cmax = jnp.max(iou_g, axis=(1, 2), keepdims=True)
        at_max = iou_g == cmax
        carg = jnp.min(
            jnp.where(at_max, idx_lin, jnp.int32(0x7FFFFFFF)),
            axis=(1, 2),
            keepdims=True,
        )
        hit = (idx_lin == carg) & (cmax >= _NEG_THR)
        forced_ref[...] = jnp.maximum(forced_ref[...], hit.astype(f32))
        return carry

    lax.fori_loop(0, _G, gstep, 0)
    run_max = max_ref[...]
    wx = wx_ref[...]
    wy = wy_ref[...]
    ww = ww_ref[...]
    wh = wh_ref[...]
    wa = wa_ref[...]
    wi = wi_ref[...]
    forced = forced_ref[...] > 0.0

    cls_t = jnp.where(run_max < _NEG_THR, 0, -1)
    cls_t = jnp.where(run_max >= _IOU_THR, 1, cls_t)
    cls_t = jnp.where(forced, 1, cls_t)
    pos = cls_t == 1
    lane_ok = idx_lin < _N
    posf = pos.astype(f32)
    validf = ((cls_t >= 0) & lane_ok).astype(f32)
    num_pos = jnp.maximum(jnp.sum(posf), 1.0)

    x = cls_ref[...]
    t = posf
    p = jax.nn.sigmoid(x)
    ce = jnp.logaddexp(0.0, x) - x * t
    p_t = p * t + (1.0 - p) * (1.0 - t)
    alpha_t = _ALPHA * t + (1.0 - _ALPHA) * (1.0 - t)
    q = 1.0 - p_t
    focal = alpha_t * ce * (q * q)
    sum_cls = jnp.sum(focal * validf)

    aw_e = aw + _EPS
    ah_e = ah + _EPS
    tgt0 = (wx - ax) / aw_e
    tgt1 = (wy - ay) / ah_e
    tgt2 = jnp.log(ww / aw_e + _EPS)
    tgt3 = jnp.log(wh / ah_e + _EPS)
    tgt4 = jnp.sin(wa - aa)
    tgt5 = jnp.cos(wa - aa)
    sum_box = 0.0
    for k, tgt in enumerate((tgt0, tgt1, tgt2, tgt3, tgt4, tgt5)):
        d = jnp.abs(box_ref[:, k] - tgt * posf)
        sl1 = jnp.where(d < _BETA, 0.5 * d * d / _BETA, d - 0.5 * _BETA)
        sum_box = sum_box + jnp.sum(sl1 * posf)

    il = il_ref[...]
    m = jnp.max(il, axis=1, keepdims=True)
    lse = m + jnp.log(jnp.sum(jnp.exp(il - m), axis=1, keepdims=True))
    picked = zero
    for k in range(_NI):
        picked = picked + jnp.where(wi == float(k), il[:, k], 0.0)
    sum_int = jnp.sum((lse[:, 0] - picked) * posf)

    out_ref[0, 0] = (
        _CLS_W * sum_cls + _BOX_W * sum_box + _INT_W * sum_int
    ) / num_pos


def kernel(cls_logits, box_preds, intention_logits, anchors, gt_boxes_xywha,
           gt_intentions):
    pad = _NP - _N
    cls_p = jnp.pad(cls_logits[..., 0], ((0, 0), (0, pad))).reshape(_B, _S, _L)
    box_p = (
        jnp.pad(box_preds, ((0, 0), (0, pad), (0, 0)))
        .transpose(0, 2, 1)
        .reshape(_B, 6, _S, _L)
    )
    il_p = (
        jnp.pad(intention_logits, ((0, 0), (0, pad), (0, 0)))
        .transpose(0, 2, 1)
        .reshape(_B, _NI, _S, _L)
    )
    anc_p = jnp.pad(anchors, ((0, pad), (0, 0))).transpose(1, 0).reshape(5, _S, _L)
    gt_all = jnp.concatenate(
        [gt_boxes_xywha, gt_intentions[..., None].astype(jnp.float32)], axis=-1
    )
    gt_p = gt_all.transpose(1, 2, 0)[..., None]

    out = pl.pallas_call(
        _loss_kernel,
        out_shape=jax.ShapeDtypeStruct((1, 1), jnp.float32),
        out_specs=pl.BlockSpec(memory_space=pltpu.SMEM),
        scratch_shapes=[pltpu.VMEM((_B, _S, _L), jnp.float32)] * 8,
    )(cls_p, box_p, il_p, anc_p, gt_p)
    return out[0, 0]
